# baseline (speedup 1.0000x reference)
import functools

import jax
import jax.numpy as jnp
from jax import lax
from jax.experimental import pallas as pl
from jax.experimental.pallas import tpu as pltpu
from jax.experimental.pallas import tpu_sc as plsc

B = 16384
EMB = 128
NW = 32
BPW = B // NW
CHUNK = 128
NCHUNK = BPW // CHUNK
LANES = 16


def _sc_gather(uids2, iids2, Ug, Ig, Um, Im):
    mesh = plsc.VectorSubcoreMesh(core_axis_name="c", subcore_axis_name="s")

    def body(u_hbm, i_hbm, ug_hbm, ig_hbm, um_hbm, im_hbm,
             out_gmf, out_um, out_im,
             uidx, iidx, bufa0, bufa1, bufb0, bufb1,
             gsem0, gsem1, wsem0, wsem1):
        wid = lax.axis_index("s") * 2 + lax.axis_index("c")
        idx_row0 = wid * NCHUNK
        pltpu.sync_copy(u_hbm.at[pl.ds(idx_row0, NCHUNK)], uidx)
        pltpu.sync_copy(i_hbm.at[pl.ds(idx_row0, NCHUNK)], iidx)

        bufa = (bufa0, bufa1)
        bufb = (bufb0, bufb1)
        gsem = (gsem0, gsem1)
        wsem = (wsem0, wsem1)
        row0 = wid * BPW

        jobs = []
        for c in range(NCHUNK):
            jobs.append((ug_hbm, ig_hbm, uidx, iidx, c, out_gmf))
        for c in range(NCHUNK):
            jobs.append((um_hbm, None, uidx, None, c, out_um))
        for c in range(NCHUNK):
            jobs.append((im_hbm, None, iidx, None, c, out_im))

        gathers = {}
        writebacks = {}

        def issue(j):
            tbl_a, tbl_b, idx_a, idx_b, c, _ = jobs[j]
            p = j % 2
            cps = [pltpu.async_copy(tbl_a.at[idx_a.at[c]], bufa[p], gsem[p])]
            if tbl_b is not None:
                cps.append(
                    pltpu.async_copy(tbl_b.at[idx_b.at[c]], bufb[p], gsem[p]))
            gathers[j] = cps

        def finish(j):
            tbl_a, tbl_b, _, _, c, out = jobs[j]
            p = j % 2
            for cp in gathers.pop(j):
                cp.wait()
            if tbl_b is not None:
                def mul_row(r, carry):
                    for k in range(EMB // LANES):
                        sl = pl.ds(k * LANES, LANES)
                        bufa[p][r, sl] = bufa[p][r, sl] * bufb[p][r, sl]
                    return carry
                lax.fori_loop(0, CHUNK, mul_row, 0, unroll=2)
            writebacks[j] = pltpu.async_copy(
                bufa[p], out.at[pl.ds(row0 + c * CHUNK, CHUNK)], wsem[p])

        njobs = len(jobs)
        issue(0)
        issue(1)
        for j in range(njobs):
            finish(j)
            if j + 2 < njobs:
                writebacks.pop(j).wait()
                issue(j + 2)
        writebacks.pop(njobs - 2).wait()
        writebacks.pop(njobs - 1).wait()

    return pl.kernel(
        body,
        out_type=[jax.ShapeDtypeStruct((B, EMB), jnp.float32)] * 3,
        mesh=mesh,
        scratch_types=[
            pltpu.VMEM((NCHUNK, CHUNK), jnp.int32),
            pltpu.VMEM((NCHUNK, CHUNK), jnp.int32),
            pltpu.VMEM((CHUNK, EMB), jnp.float32),
            pltpu.VMEM((CHUNK, EMB), jnp.float32),
            pltpu.VMEM((CHUNK, EMB), jnp.float32),
            pltpu.VMEM((CHUNK, EMB), jnp.float32),
            pltpu.SemaphoreType.DMA,
            pltpu.SemaphoreType.DMA,
            pltpu.SemaphoreType.DMA,
            pltpu.SemaphoreType.DMA,
        ],
    )(uids2, iids2, Ug, Ig, Um, Im)


def _mlp_body(gmf_r, um_r, im_r, w1a_r, w1b_r, b1_r, w2_r, b2_r,
              w3_r, b3_r, wfa_r, wfb_r, bf_r, out_r):
    bf16 = jnp.bfloat16
    f32 = jnp.float32
    h1 = (jnp.dot(um_r[...].astype(bf16), w1a_r[...].astype(bf16),
                  preferred_element_type=f32)
          + jnp.dot(im_r[...].astype(bf16), w1b_r[...].astype(bf16),
                    preferred_element_type=f32)
          + b1_r[...])
    h1 = jnp.maximum(h1, 0.0).astype(bf16)
    h2 = jnp.maximum(
        jnp.dot(h1, w2_r[...].astype(bf16), preferred_element_type=f32)
        + b2_r[...], 0.0).astype(bf16)
    h3 = jnp.maximum(
        jnp.dot(h2, w3_r[...].astype(bf16), preferred_element_type=f32)
        + b3_r[...], 0.0)
    z = (jnp.sum(gmf_r[...] * wfa_r[...], axis=1)
         + jnp.sum(h3 * wfb_r[...], axis=1) + bf_r[0, 0])
    out_r[...] = 1.0 / (1.0 + jnp.exp(-z))


def _tc_mlp(gmf, um, im, W1a, W1b, b1, W2, b2, W3, b3, wfa_row, wfb_row, bf):
    R = 1024
    grid = (B // R,)
    row_spec = pl.BlockSpec((R, EMB), lambda i: (i, 0))

    def fixed(shape):
        return pl.BlockSpec(shape, lambda i: tuple(0 for _ in shape))

    return pl.pallas_call(
        _mlp_body,
        grid=grid,
        in_specs=[
            row_spec, row_spec, row_spec,
            fixed((EMB, 256)), fixed((EMB, 256)), fixed((1, 256)),
            fixed((256, 128)), fixed((1, 128)),
            fixed((128, 64)), fixed((1, 64)),
            fixed((1, EMB)), fixed((1, 64)), fixed((1, 1)),
        ],
        out_specs=pl.BlockSpec((R,), lambda i: (i,)),
        out_shape=jax.ShapeDtypeStruct((B,), jnp.float32),
    )(gmf, um, im, W1a, W1b, b1, W2, b2, W3, b3, wfa_row, wfb_row, bf)


def kernel(user_ids, item_ids, Ug, Ig, Um, Im, W1, b1, W2, b2, W3, b3, Wf, bf):
    uids2 = user_ids.astype(jnp.int32).reshape(B // CHUNK, CHUNK)
    iids2 = item_ids.astype(jnp.int32).reshape(B // CHUNK, CHUNK)
    gmf, um, im = _sc_gather(uids2, iids2, Ug, Ig, Um, Im)
    W1a, W1b = W1[:EMB], W1[EMB:]
    wfa_row = Wf[:EMB].reshape(1, EMB)
    wfb_row = Wf[EMB:].reshape(1, 64)
    out = _tc_mlp(gmf, um, im,
                  W1a, W1b, b1.reshape(1, -1),
                  W2, b2.reshape(1, -1), W3, b3.reshape(1, -1),
                  wfa_row, wfb_row, bf.reshape(1, 1))
    return out.reshape(B, 1)
